# baseline (speedup 1.0000x reference)
import jax
import jax.numpy as jnp
from jax.experimental import pallas as pl

_WAY = 5


def _mul_dup_body(x_ref, s_ref, e_ref, o_ref):
    scale = jnp.dot(s_ref[...], e_ref[...],
                    preferred_element_type=jnp.float32)
    y = x_ref[...] * scale
    o_ref[0, 0] = y
    o_ref[0, 1] = y


def kernel(x, s_ca, shuffle_num):
    N, c, h, w = x.shape
    hw = h * w
    CG = 32
    L = CG * hw
    RPS = c // CG
    G = N // _WAY
    ROWS_W = G * RPS
    R = 192
    nblk = (N * RPS) // R
    bpw = ROWS_W // R

    x2 = x.reshape(N * RPS, L)
    s2 = s_ca.reshape(N * RPS, CG)
    E = jnp.repeat(jnp.eye(CG, dtype=x.dtype), hw, axis=1)

    out = pl.pallas_call(
        _mul_dup_body,
        grid=(nblk,),
        in_specs=[
            pl.BlockSpec((R, L), lambda i: (i, 0)),
            pl.BlockSpec((R, CG), lambda i: (i, 0)),
            pl.BlockSpec((CG, L), lambda i: (0, 0)),
        ],
        out_specs=pl.BlockSpec((1, 2, R, L),
                               lambda i: (i // bpw, 0, i % bpw, 0)),
        out_shape=jax.ShapeDtypeStruct((_WAY, 2, ROWS_W, L), x.dtype),
    )(x2, s2, E)
    return out.reshape(2 * N, c, h, w)


# speedup vs baseline: 5.4764x; 5.4764x over previous
import jax
import jax.numpy as jnp
from jax.experimental import pallas as pl

_WAY = 5


def _mul_dup_body(x_ref, s_ref, o_ref):
    y = x_ref[...] * s_ref[...]
    o_ref[:, 0] = y[None]
    o_ref[:, 1] = y[None]


def kernel(x, s_ca, shuffle_num):
    N, c, h, w = x.shape
    hw = h * w
    G = N // _WAY
    B = 8
    bpw = G // B

    x3 = x.reshape(N, c, hw)
    s3 = s_ca.reshape(N, c, 1)

    out = pl.pallas_call(
        _mul_dup_body,
        grid=(N // B,),
        in_specs=[
            pl.BlockSpec((B, c, hw), lambda i: (i, 0, 0)),
            pl.BlockSpec((B, c, 1), lambda i: (i, 0, 0)),
        ],
        out_specs=pl.BlockSpec((1, 2, B, c, hw),
                               lambda i: (i // bpw, 0, i % bpw, 0, 0)),
        out_shape=jax.ShapeDtypeStruct((_WAY, 2, G, c, hw), x.dtype),
    )(x3, s3)
    return out.reshape(2 * N, c, h, w)


# speedup vs baseline: 6.0004x; 1.0957x over previous
import jax
import jax.numpy as jnp
from jax.experimental import pallas as pl
import jax.experimental.pallas.tpu as pltpu

_WAY = 5
_B = 8
_D = 3


def _body(x_hbm, s_hbm, o_hbm, xb, sb, yb, in_sem, s_sem, out_sem):
    i = pl.program_id(0)
    S = pl.num_programs(0)
    G = x_hbm.shape[0] // _WAY
    bpw = G // _B
    slot = jax.lax.rem(i, 2)
    nslot = jax.lax.rem(i + 1, 2)
    yslot = jax.lax.rem(i, _D)

    def start_in(step, sl):
        pltpu.make_async_copy(x_hbm.at[pl.ds(step * _B, _B)],
                              xb.at[sl], in_sem.at[sl]).start()
        pltpu.make_async_copy(s_hbm.at[pl.ds(step * _B, _B)],
                              sb.at[sl], s_sem.at[sl]).start()

    @pl.when(i == 0)
    def _():
        start_in(i, slot)

    @pl.when(i + 1 < S)
    def _():
        start_in(i + 1, nslot)

    pltpu.make_async_copy(x_hbm.at[pl.ds(i * _B, _B)],
                          xb.at[slot], in_sem.at[slot]).wait()
    pltpu.make_async_copy(s_hbm.at[pl.ds(i * _B, _B)],
                          sb.at[slot], s_sem.at[slot]).wait()

    @pl.when(i >= _D)
    def _():
        for j in range(2):
            pltpu.make_async_copy(yb.at[yslot], o_hbm.at[0, j, pl.ds(0, _B)],
                                  out_sem.at[yslot, j]).wait()

    st = jnp.swapaxes(sb[slot], 0, 1)
    for b in range(_B):
        yb[yslot, b] = xb[slot, b] * st[:, b][:, None]

    way = i // bpw
    g0 = jax.lax.rem(i, bpw) * _B
    for j in range(2):
        pltpu.make_async_copy(yb.at[yslot], o_hbm.at[way, j, pl.ds(g0, _B)],
                              out_sem.at[yslot, j]).start()

    @pl.when(i == S - 1)
    def _():
        for k in range(_D):
            for j in range(2):
                pltpu.make_async_copy(yb.at[k],
                                      o_hbm.at[0, j, pl.ds(0, _B)],
                                      out_sem.at[k, j]).wait()


def kernel(x, s_ca, shuffle_num):
    N, c, h, w = x.shape
    hw = h * w
    G = N // _WAY

    x3 = x.reshape(N, c, hw)
    s2 = s_ca.reshape(N, c)

    out = pl.pallas_call(
        _body,
        grid=(N // _B,),
        in_specs=[
            pl.BlockSpec(memory_space=pl.ANY),
            pl.BlockSpec(memory_space=pl.ANY),
        ],
        out_specs=pl.BlockSpec(memory_space=pl.ANY),
        out_shape=jax.ShapeDtypeStruct((_WAY, 2, G, c, hw), x.dtype),
        scratch_shapes=[
            pltpu.VMEM((2, _B, c, hw), x.dtype),
            pltpu.VMEM((2, _B, c), x.dtype),
            pltpu.VMEM((_D, _B, c, hw), x.dtype),
            pltpu.SemaphoreType.DMA((2,)),
            pltpu.SemaphoreType.DMA((2,)),
            pltpu.SemaphoreType.DMA((_D, 2)),
        ],
    )(x3, s2)
    return out.reshape(2 * N, c, h, w)
